# Initial kernel scaffold; baseline (speedup 1.0000x reference)
import jax
import jax.numpy as jnp
from jax.experimental import pallas as pl


def kernel(vertices, params):
    raise NotImplementedError("write your pallas kernel here")


# speedup vs baseline: 2.7268x; 2.7268x over previous
import functools

import numpy as np
import jax
import jax.numpy as jnp
from jax import lax
from jax.experimental import pallas as pl
from jax.experimental.pallas import tpu as pltpu

NEI = 32
POOL_NEI = 4
PAD3 = 16


def _knn_body(q_ref, rt_ref, o_ref, *, K, Vc):
    qb = q_ref[0]
    rt = rt_ref[0]
    RB = qb.shape[0]
    inner = jnp.dot(qb[:, :3], rt, preferred_element_type=jnp.float32)
    qc = rt[0:1, :] ** 2 + rt[1:2, :] ** 2 + rt[2:3, :] ** 2
    qr = (qb[:, 0:1] ** 2 + qb[:, 1:2] ** 2 + qb[:, 2:3] ** 2)
    d = (-2.0 * inner + qc) + qr
    iota = lax.broadcasted_iota(jnp.int32, (RB, Vc), 1)
    cols = []
    for t in range(K + 1):
        m = jnp.min(d, axis=1, keepdims=True)
        am = jnp.min(jnp.where(d == m, iota, Vc), axis=1, keepdims=True)
        if t > 0:
            cols.append(am)
        d = jnp.where(iota == am, jnp.inf, d)
    o_ref[0] = jnp.concatenate(cols, axis=1)


def _knn(q, r, K):
    B, R, _ = q.shape
    Vc = r.shape[1]
    rt = jnp.transpose(r, (0, 2, 1))
    RB = min(R, 256)
    grid = (B, R // RB)
    return pl.pallas_call(
        functools.partial(_knn_body, K=K, Vc=Vc),
        grid=grid,
        in_specs=[
            pl.BlockSpec((1, RB, PAD3), lambda b, i: (b, i, 0)),
            pl.BlockSpec((1, 3, Vc), lambda b, i: (b, 0, 0)),
        ],
        out_specs=pl.BlockSpec((1, RB, K), lambda b, i: (b, i, 0)),
        out_shape=jax.ShapeDtypeStruct((B, R, K), jnp.int32),
    )(q, rt)


def _gather_body(idx_ref, tab_ref, o_ref, *, Vt):
    ib = idx_ref[0]
    tab = tab_ref[0]
    GB = ib.shape[1]
    ohT = (lax.broadcasted_iota(jnp.int32, (Vt, GB), 0) == ib).astype(jnp.float32)
    o_ref[0] = lax.dot_general(ohT, tab, (((0,), (0,)), ((), ())),
                               preferred_element_type=jnp.float32,
                               precision=lax.Precision.HIGHEST)


def _gather_rows(table, idx):
    B, Vt, D = table.shape
    Rg = idx.shape[1]
    GB = min(Rg, 512)
    NB = Rg // GB
    idx3 = idx.reshape(B * NB, 1, GB)
    return pl.pallas_call(
        functools.partial(_gather_body, Vt=Vt),
        grid=(B, NB),
        in_specs=[
            pl.BlockSpec((1, 1, GB), lambda b, i: (b * NB + i, 0, 0)),
            pl.BlockSpec((1, Vt, D), lambda b, i: (b, 0, 0)),
        ],
        out_specs=pl.BlockSpec((1, GB, D), lambda b, i: (b, i, 0)),
        out_shape=jax.ShapeDtypeStruct((B, Rg, D), jnp.float32),
    )(idx3, table)


def _ndn_body(gv_ref, c_ref, o_ref, *, K):
    gv = gv_ref[0]
    ce = c_ref[0]
    VB = ce.shape[0]
    nd = gv.reshape(VB, K, PAD3) - ce[:, None, :]
    n2 = jnp.sum(nd * nd, axis=-1, keepdims=True)
    denom = jnp.maximum(jnp.sqrt(n2), 1e-12)
    o_ref[0] = (nd / denom).reshape(VB * K, PAD3)


def _ndn(gv, centers, K):
    B, V, _ = centers.shape
    VB = min(V, 128)
    return pl.pallas_call(
        functools.partial(_ndn_body, K=K),
        grid=(B, V // VB),
        in_specs=[
            pl.BlockSpec((1, VB * K, PAD3), lambda b, i: (b, i, 0)),
            pl.BlockSpec((1, VB, PAD3), lambda b, i: (b, i, 0)),
        ],
        out_specs=pl.BlockSpec((1, VB * K, PAD3), lambda b, i: (b, i, 0)),
        out_shape=jax.ShapeDtypeStruct((B, V * K, PAD3), jnp.float32),
    )(gv, centers)


def _conv0_body(ndn_ref, sdn_ref, o_ref, *, K):
    ndn = ndn_ref[0]
    sdn = sdn_ref[...]
    VB = ndn.shape[0] // K
    th = jax.nn.relu(jnp.dot(ndn, sdn, preferred_element_type=jnp.float32))
    o_ref[0] = jnp.max(th.reshape(VB, K, sdn.shape[1]), axis=1)


def _conv_surface(ndn, sdn, V):
    B = ndn.shape[0]
    C = sdn.shape[1]
    VB = min(V, 128)
    return pl.pallas_call(
        functools.partial(_conv0_body, K=NEI),
        grid=(B, V // VB),
        in_specs=[
            pl.BlockSpec((1, VB * NEI, PAD3), lambda b, i: (b, i, 0)),
            pl.BlockSpec((PAD3, C), lambda b, i: (0, 0)),
        ],
        out_specs=pl.BlockSpec((1, VB, C), lambda b, i: (b, i, 0)),
        out_shape=jax.ShapeDtypeStruct((B, V, C), jnp.float32),
    )(ndn, sdn)


def _linear_body(x_ref, w_ref, b_ref, o_ref):
    o_ref[0] = (jnp.dot(x_ref[0], w_ref[...],
                        preferred_element_type=jnp.float32) + b_ref[...])


def _linear(x, w, b):
    B, V, Cin = x.shape
    Cout = w.shape[1]
    MB = min(V, 256)
    return pl.pallas_call(
        _linear_body,
        grid=(B, V // MB),
        in_specs=[
            pl.BlockSpec((1, MB, Cin), lambda b, i: (b, i, 0)),
            pl.BlockSpec((Cin, Cout), lambda b, i: (0, 0)),
            pl.BlockSpec((1, Cout), lambda b, i: (0, 0)),
        ],
        out_specs=pl.BlockSpec((1, MB, Cout), lambda b, i: (b, i, 0)),
        out_shape=jax.ShapeDtypeStruct((B, V, Cout), jnp.float32),
    )(x, w, b)


def _combine_body(ndn_ref, gs_ref, cf_ref, sdn_ref, o_ref, *, K, relu):
    ndn = ndn_ref[0]
    gs = gs_ref[0]
    cf = cf_ref[0]
    sdn = sdn_ref[...]
    VB, C = cf.shape
    th = jax.nn.relu(jnp.dot(ndn, sdn, preferred_element_type=jnp.float32))
    act = jnp.max((th * gs).reshape(VB, K, C), axis=1)
    out = cf + act
    if relu:
        out = jax.nn.relu(out)
    o_ref[0] = out


def _conv_combine(ndn, gs, cf, sdn, relu):
    B, V, C = cf.shape
    VB = min(V, 128 if C <= 256 else 32)
    return pl.pallas_call(
        functools.partial(_combine_body, K=NEI, relu=relu),
        grid=(B, V // VB),
        in_specs=[
            pl.BlockSpec((1, VB * NEI, PAD3), lambda b, i: (b, i, 0)),
            pl.BlockSpec((1, VB * NEI, C), lambda b, i: (b, i, 0)),
            pl.BlockSpec((1, VB, C), lambda b, i: (b, i, 0)),
            pl.BlockSpec((PAD3, C), lambda b, i: (0, 0)),
        ],
        out_specs=pl.BlockSpec((1, VB, C), lambda b, i: (b, i, 0)),
        out_shape=jax.ShapeDtypeStruct((B, V, C), jnp.float32),
    )(ndn, gs, cf, sdn)


def _pool_body(g_ref, o_ref, *, K):
    g = g_ref[0]
    C = g.shape[1]
    PB = g.shape[0] // K
    o_ref[0] = jnp.max(g.reshape(PB, K, C), axis=1)


def _pool_max(g, R, K):
    B, _, C = g.shape
    PB = min(R, 128)
    return pl.pallas_call(
        functools.partial(_pool_body, K=K),
        grid=(B, R // PB),
        in_specs=[pl.BlockSpec((1, PB * K, C), lambda b, i: (b, i, 0))],
        out_specs=pl.BlockSpec((1, PB, C), lambda b, i: (b, i, 0)),
        out_shape=jax.ShapeDtypeStruct((B, R, C), jnp.float32),
    )(g)


def _psum_rows(p):
    R = p.shape[0]
    while R > 1:
        R //= 2
        p = p[:R] + p[R:]
    return p


def _psum_lanes(p):
    N = p.shape[1]
    while N > 1:
        N //= 2
        p = p[:, :N] + p[:, N:]
    return p


def _sa_A(x, qkw):
    qk = jnp.dot(qkw, x, preferred_element_type=jnp.float32)
    e = lax.dot_general(qk, qk, (((0,), (0,)), ((), ())),
                        preferred_element_type=jnp.float32)
    m = jnp.max(e, axis=1, keepdims=True)
    p = jnp.exp(e - m)
    s = _psum_lanes(p)
    return p / s


def _sa_cs_body(x_ref, qkw_ref, cs_ref):
    A = _sa_A(x_ref[0], qkw_ref[...])
    cs_ref[0] = _psum_rows(A)


def _sa_t_body(x_ref, qkw_ref, vw_ref, vb_ref, cs_ref, tw_ref, tb_ref, t_ref):
    x = x_ref[0]
    v = jnp.dot(vw_ref[...], x, preferred_element_type=jnp.float32) + vb_ref[...]
    An = _sa_A(x, qkw_ref[...]) / (1e-9 + cs_ref[0])
    x_r = jnp.dot(v, An, preferred_element_type=jnp.float32)
    t_ref[0] = (jnp.dot(tw_ref[...], x - x_r,
                        preferred_element_type=jnp.float32) + tb_ref[...])


def _bn02_sum(t):
    Bc = t.shape[0]
    while Bc > 1:
        Bc //= 2
        t = t[:Bc] + t[Bc:]
    return _psum_lanes(t[0])[None]


def _sa_out_body(x_ref, t_ref, g_ref, b_ref, o_ref):
    t = t_ref[...]
    cnt = t.shape[0] * t.shape[2]
    mean = _bn02_sum(t) / cnt
    var = _bn02_sum((t - mean) ** 2) / cnt
    tn = (t - mean) / jnp.sqrt(var + 1e-5) * g_ref[...] + b_ref[...]
    o_ref[...] = x_ref[...] + jax.nn.relu(tn)


def _sa_layer(p, x):
    B, C, N = x.shape
    C4 = C // 4
    cs = pl.pallas_call(
        _sa_cs_body,
        grid=(B,),
        in_specs=[
            pl.BlockSpec((1, C, N), lambda b: (b, 0, 0)),
            pl.BlockSpec((C4, C), lambda b: (0, 0)),
        ],
        out_specs=pl.BlockSpec((1, 1, N), lambda b: (b, 0, 0)),
        out_shape=jax.ShapeDtypeStruct((B, 1, N), jnp.float32),
    )(x, p['qk_w'])
    t = pl.pallas_call(
        _sa_t_body,
        grid=(B,),
        in_specs=[
            pl.BlockSpec((1, C, N), lambda b: (b, 0, 0)),
            pl.BlockSpec((C4, C), lambda b: (0, 0)),
            pl.BlockSpec((C, C), lambda b: (0, 0)),
            pl.BlockSpec((C, 1), lambda b: (0, 0)),
            pl.BlockSpec((1, 1, N), lambda b: (b, 0, 0)),
            pl.BlockSpec((C, C), lambda b: (0, 0)),
            pl.BlockSpec((C, 1), lambda b: (0, 0)),
        ],
        out_specs=pl.BlockSpec((1, C, N), lambda b: (b, 0, 0)),
        out_shape=jax.ShapeDtypeStruct((B, C, N), jnp.float32),
    )(x, p['qk_w'], p['v_w'], p['v_b'].reshape(C, 1), cs,
      p['t_w'], p['t_b'].reshape(C, 1))
    return pl.pallas_call(
        _sa_out_body,
        grid=(1,),
        in_specs=[
            pl.BlockSpec((B, C, N), lambda i: (0, 0, 0)),
            pl.BlockSpec((B, C, N), lambda i: (0, 0, 0)),
            pl.BlockSpec((C, 1), lambda i: (0, 0)),
            pl.BlockSpec((C, 1), lambda i: (0, 0)),
        ],
        out_specs=pl.BlockSpec((B, C, N), lambda i: (0, 0, 0)),
        out_shape=jax.ShapeDtypeStruct((B, C, N), jnp.float32),
    )(x, t, p['bn_g'].reshape(C, 1), p['bn_b'].reshape(C, 1))


def _cls_body(fm_ref, w1_ref, b1_ref, g_ref, be_ref, w2_ref, b2_ref, o_ref):
    fg = jnp.max(fm_ref[...], axis=1)
    h = lax.dot_general(fg, w1_ref[...], (((1,), (1,)), ((), ())),
                        preferred_element_type=jnp.float32) + b1_ref[...]
    mean = jnp.mean(h, axis=0, keepdims=True)
    var = jnp.mean((h - mean) ** 2, axis=0, keepdims=True)
    h = (h - mean) / jnp.sqrt(var + 1e-5) * g_ref[...] + be_ref[...]
    h = jax.nn.relu(h)
    o_ref[...] = lax.dot_general(h, w2_ref[...], (((1,), (1,)), ((), ())),
                                 preferred_element_type=jnp.float32) + b2_ref[...]


def _classifier(fm4, c):
    B = fm4.shape[0]
    return pl.pallas_call(
        _cls_body,
        grid=(1,),
        in_specs=[
            pl.BlockSpec(fm4.shape, lambda i: (0, 0, 0)),
            pl.BlockSpec(c['w1'].shape, lambda i: (0, 0)),
            pl.BlockSpec((1, 256), lambda i: (0, 0)),
            pl.BlockSpec((1, 256), lambda i: (0, 0)),
            pl.BlockSpec((1, 256), lambda i: (0, 0)),
            pl.BlockSpec(c['w2'].shape, lambda i: (0, 0)),
            pl.BlockSpec((1, 40), lambda i: (0, 0)),
        ],
        out_specs=pl.BlockSpec((B, 40), lambda i: (0, 0)),
        out_shape=jax.ShapeDtypeStruct((B, 40), jnp.float32),
    )(fm4, c['w1'], c['b1'].reshape(1, 256), c['bn_g'].reshape(1, 256),
      c['bn_b'].reshape(1, 256), c['w2'], c['b2'].reshape(1, 40))


def _pad3(x):
    B, V, _ = x.shape
    return jnp.concatenate(
        [x, jnp.zeros((B, V, PAD3 - 3), jnp.float32)], axis=2)


def _norm_dirs(d):
    n = jnp.linalg.norm(d, axis=0, keepdims=True)
    d = d / jnp.maximum(n, 1e-12)
    return jnp.concatenate(
        [d, jnp.zeros((PAD3 - 3, d.shape[1]), jnp.float32)], axis=0)


def _conv_layer(ndn, fm, nidx, p, out_c, relu):
    feat = _linear(fm, p['weights'], p['bias'].reshape(1, -1))
    center = feat[:, :, :out_c]
    supp = feat[:, :, out_c:]
    gs = _gather_rows(supp, nidx)
    return _conv_combine(ndn, gs, center, _norm_dirs(p['directions']), relu)


def kernel(vertices, params):
    B, V, _ = vertices.shape
    vp = _pad3(vertices)

    ni1 = _knn(vp, vertices, NEI)
    ni1f = ni1.reshape(B, V * NEI)
    gv1 = _gather_rows(vp, ni1f)
    ndn1 = _ndn(gv1, vp, NEI)

    fm0 = _conv_surface(ndn1, _norm_dirs(params['conv_0']['directions']), V)
    fm1 = _conv_layer(ndn1, fm0, ni1f, params['conv_1'], 64, True)
    fm1 = jnp.transpose(_sa_layer(params['sa1'],
                                  jnp.transpose(fm1, (0, 2, 1))), (0, 2, 1))

    s1 = np.random.RandomState(1).permutation(V)[:V // 4]
    q1 = vertices[:, s1, :]
    q1p = vp[:, s1, :]
    nip1 = _knn(q1p, vertices, POOL_NEI)
    gp1 = _gather_rows(fm1, nip1.reshape(B, -1))
    fm1p = _pool_max(gp1, V // 4, POOL_NEI)

    V1 = V // 4
    ni2 = _knn(q1p, q1, NEI)
    ni2f = ni2.reshape(B, V1 * NEI)
    gv2 = _gather_rows(q1p, ni2f)
    ndn2 = _ndn(gv2, q1p, NEI)

    fm2 = _conv_layer(ndn2, fm1p, ni2f, params['conv_2'], 128, True)
    fm2 = jnp.transpose(_sa_layer(params['sa2'],
                                  jnp.transpose(fm2, (0, 2, 1))), (0, 2, 1))
    fm3 = _conv_layer(ndn2, fm2, ni2f, params['conv_3'], 256, True)
    fm3 = jnp.transpose(_sa_layer(params['sa3'],
                                  jnp.transpose(fm3, (0, 2, 1))), (0, 2, 1))

    s2 = np.random.RandomState(2).permutation(V1)[:V1 // 4]
    q2 = q1[:, s2, :]
    q2p = q1p[:, s2, :]
    nip2 = _knn(q2p, q1, POOL_NEI)
    gp2 = _gather_rows(fm3, nip2.reshape(B, -1))
    fm3p = _pool_max(gp2, V1 // 4, POOL_NEI)

    V2 = V1 // 4
    ni3 = _knn(q2p, q2, NEI)
    ni3f = ni3.reshape(B, V2 * NEI)
    gv3 = _gather_rows(q2p, ni3f)
    ndn3 = _ndn(gv3, q2p, NEI)

    fm4 = _conv_layer(ndn3, fm3p, ni3f, params['conv_4'], 1024, False)
    fm4 = jnp.transpose(_sa_layer(params['sa4'],
                                  jnp.transpose(fm4, (0, 2, 1))), (0, 2, 1))

    return _classifier(fm4, params['cls'])
